# Initial kernel scaffold; baseline (speedup 1.0000x reference)
import jax
import jax.numpy as jnp
from jax.experimental import pallas as pl


def kernel(x, edge_index, fc_W, fc_b, conv_mu_W, conv_mu_b, conv_logstd_W, conv_logstd_b, addon_mu_W, addon_mu_b, addon_logstd_W, addon_logstd_b):
    raise NotImplementedError("write your pallas kernel here")


# speedup vs baseline: 6.1697x; 6.1697x over previous
import functools

import jax
import jax.numpy as jnp
from jax.experimental import pallas as pl
from jax.experimental.pallas import tpu as pltpu

_N = 10000
_RB = 1000


def _proj_body(dis_ref, s_ref, hs_ref, wmu_ref, wls_ref, bmu_ref, bls_ref,
               mu_ref, ls_ref):
    agg = dis_ref[...] * (s_ref[...] + hs_ref[...])
    mu_ref[...] = jnp.dot(agg, wmu_ref[...],
                          preferred_element_type=jnp.float32) + bmu_ref[...]
    ls_ref[...] = jnp.dot(agg, wls_ref[...],
                          preferred_element_type=jnp.float32) + bls_ref[...]


@jax.jit
def _project(dis, s, hs, wmu_t, wls_t, bmu, bls):
    grid = _N // _RB
    return pl.pallas_call(
        _proj_body,
        grid=(grid,),
        in_specs=[
            pl.BlockSpec((_RB, 1), lambda i: (i, 0)),
            pl.BlockSpec((_RB, 128), lambda i: (i, 0)),
            pl.BlockSpec((_RB, 128), lambda i: (i, 0)),
            pl.BlockSpec((128, 256), lambda i: (0, 0)),
            pl.BlockSpec((128, 256), lambda i: (0, 0)),
            pl.BlockSpec((1, 256), lambda i: (0, 0)),
            pl.BlockSpec((1, 256), lambda i: (0, 0)),
        ],
        out_specs=[
            pl.BlockSpec((_RB, 256), lambda i: (i, 0)),
            pl.BlockSpec((_RB, 256), lambda i: (i, 0)),
        ],
        out_shape=[
            jax.ShapeDtypeStruct((_N, 256), jnp.float32),
            jax.ShapeDtypeStruct((_N, 256), jnp.float32),
        ],
    )(dis, s, hs, wmu_t, wls_t, bmu, bls)


def kernel(x, edge_index, fc_W, fc_b, conv_mu_W, conv_mu_b, conv_logstd_W,
           conv_logstd_b, addon_mu_W, addon_mu_b, addon_logstd_W,
           addon_logstd_b):
    src = edge_index[0]
    dst = edge_index[1]
    n = x.shape[0]

    hidden = jax.nn.relu(x @ fc_W.T + fc_b)
    deg = jnp.zeros((n,), jnp.float32).at[dst].add(1.0) + 1.0
    dis = jax.lax.rsqrt(deg)
    hs = hidden * dis[:, None]
    s = jnp.zeros_like(hs).at[dst].add(hs[src])

    wmu_t = jnp.pad(jnp.concatenate([conv_mu_W, addon_mu_W], 0),
                    ((0, 28), (0, 0))).T
    wls_t = jnp.pad(jnp.concatenate([conv_logstd_W, addon_logstd_W], 0),
                    ((0, 28), (0, 0))).T
    bmu = jnp.pad(jnp.concatenate([conv_mu_b, addon_mu_b]), (0, 28))[None, :]
    bls = jnp.pad(jnp.concatenate([conv_logstd_b, addon_logstd_b]),
                  (0, 28))[None, :]

    mu, ls = _project(dis[:, None], s, hs, wmu_t, wls_t, bmu, bls)
    return mu[:, :228], ls[:, :228]


# speedup vs baseline: 23.8857x; 3.8715x over previous
import functools

import jax
import jax.numpy as jnp
from jax import lax
from jax.experimental import pallas as pl
from jax.experimental.pallas import tpu as pltpu
from jax.experimental.pallas import tpu_sc as plsc

_N = 10000
_NP = 10240
_D = 128
_E = 320000
_NW = 32
_CH = 128
_NCH = 79
_EPT = _CH * _NCH
_EP = _EPT * _NW
_RPT = _NP // 16
_RB = 1024


def _sc_mesh():
    return plsc.VectorSubcoreMesh(core_axis_name="c", subcore_axis_name="s")


def _deg_body(edge_hbm, out_hbm, idx_v, hist_v):
    c = lax.axis_index("c")
    s = lax.axis_index("s")
    w = c * 16 + s
    zero16 = jnp.zeros((16,), jnp.float32)
    ones16 = jnp.ones((16,), jnp.float32)

    def zbody(i, carry):
        hist_v[pl.ds(pl.multiple_of(i * 16, 16), 16)] = zero16
        return carry

    lax.fori_loop(0, _NP // 16, zbody, 0)

    ebase = pl.multiple_of(w * _EPT, 8)
    pltpu.sync_copy(edge_hbm.at[1, pl.ds(ebase, _EPT)], idx_v)

    def body(i, carry):
        idx = idx_v[pl.ds(pl.multiple_of(i * 16, 16), 16)]
        plsc.addupdate_scatter(hist_v, [idx], ones16)
        return carry

    lax.fori_loop(0, _EPT // 16, body, 0)
    pltpu.sync_copy(hist_v, out_hbm.at[w])


@jax.jit
def _degrees(edge_pad):
    return pl.kernel(
        _deg_body,
        out_type=jax.ShapeDtypeStruct((_NW, _NP), jnp.float32),
        mesh=_sc_mesh(),
        scratch_types=[
            pltpu.VMEM((_EPT,), jnp.int32),
            pltpu.VMEM((_NP,), jnp.float32),
        ],
        compiler_params=pltpu.CompilerParams(needs_layout_passes=False),
    )(edge_pad)


def _agg_body(hs_hbm, edge_hbm, out_hbm, acc, zbuf, rows, srcb, dstb, gsem):
    c = lax.axis_index("c")
    s = lax.axis_index("s")
    w = c * 16 + s

    zero16 = jnp.zeros((16,), jnp.float32)
    for r in range(16):
        for k in range(8):
            zbuf[r, pl.ds(k * 16, 16)] = zero16

    rbase = pl.multiple_of(s * _RPT, 8)

    def zbody(i, carry):
        pltpu.sync_copy(
            zbuf, acc.at[pl.ds(pl.multiple_of(s * _RPT + i * 16, 16), 16)])
        return carry

    lax.fori_loop(0, _RPT // 16, zbody, 0)
    plsc.subcore_barrier()

    ebase = pl.multiple_of(w * _EPT, 8)

    def body(j, carry):
        off = pl.multiple_of(ebase + j * _CH, 8)
        pltpu.sync_copy(edge_hbm.at[0, pl.ds(off, _CH)], srcb)
        pltpu.sync_copy(edge_hbm.at[1, pl.ds(off, _CH)], dstb)
        pltpu.async_copy(hs_hbm.at[srcb], rows, gsem).wait()
        pltpu.sync_copy(rows, acc.at[dstb], add=True)
        return carry

    lax.fori_loop(0, _NCH, body, 0)
    plsc.subcore_barrier()
    pltpu.sync_copy(acc.at[pl.ds(rbase, _RPT)],
                    out_hbm.at[c, pl.ds(rbase, _RPT)])


@jax.jit
def _aggregate(hs_pad, edge_pad):
    return pl.kernel(
        _agg_body,
        out_type=jax.ShapeDtypeStruct((2, _NP, _D), jnp.float32),
        mesh=_sc_mesh(),
        scratch_types=[
            pltpu.VMEM_SHARED((_NP, _D), jnp.float32),
            pltpu.VMEM((16, _D), jnp.float32),
            pltpu.VMEM((_CH, _D), jnp.float32),
            pltpu.VMEM((_CH,), jnp.int32),
            pltpu.VMEM((_CH,), jnp.int32),
            pltpu.SemaphoreType.DMA,
        ],
        compiler_params=pltpu.CompilerParams(needs_layout_passes=False),
    )(hs_pad, edge_pad)


def _hs_body(degp_ref, x_ref, w_ref, b_ref, hs_ref, dis_ref):
    deg = jnp.sum(degp_ref[...], axis=0, keepdims=True) + 1.0
    dis = lax.rsqrt(deg).T
    h = jnp.maximum(
        jnp.dot(x_ref[...], w_ref[...], preferred_element_type=jnp.float32)
        + b_ref[...], 0.0)
    hs_ref[...] = h * dis
    dis_ref[...] = dis


@jax.jit
def _hidden_scaled(deg_parts, x_pad, fc_Wt, fc_b2):
    return pl.pallas_call(
        _hs_body,
        grid=(_NP // _RB,),
        in_specs=[
            pl.BlockSpec((_NW, _RB), lambda i: (0, i)),
            pl.BlockSpec((_RB, _D), lambda i: (i, 0)),
            pl.BlockSpec((_D, _D), lambda i: (0, 0)),
            pl.BlockSpec((1, _D), lambda i: (0, 0)),
        ],
        out_specs=[
            pl.BlockSpec((_RB, _D), lambda i: (i, 0)),
            pl.BlockSpec((_RB, 1), lambda i: (i, 0)),
        ],
        out_shape=[
            jax.ShapeDtypeStruct((_NP, _D), jnp.float32),
            jax.ShapeDtypeStruct((_NP, 1), jnp.float32),
        ],
    )(deg_parts, x_pad, fc_Wt, fc_b2)


def _proj_body(dis_ref, sp_ref, hs_ref, wmu_ref, wls_ref, bmu_ref, bls_ref,
               mu_ref, ls_ref):
    ssum = sp_ref[0] + sp_ref[1]
    agg = dis_ref[...] * (ssum + hs_ref[...])
    mu_ref[...] = jnp.dot(agg, wmu_ref[...],
                          preferred_element_type=jnp.float32) + bmu_ref[...]
    ls_ref[...] = jnp.dot(agg, wls_ref[...],
                          preferred_element_type=jnp.float32) + bls_ref[...]


@jax.jit
def _project(dis, s_parts, hs, wmu_t, wls_t, bmu, bls):
    return pl.pallas_call(
        _proj_body,
        grid=(_NP // _RB,),
        in_specs=[
            pl.BlockSpec((_RB, 1), lambda i: (i, 0)),
            pl.BlockSpec((2, _RB, _D), lambda i: (0, i, 0)),
            pl.BlockSpec((_RB, _D), lambda i: (i, 0)),
            pl.BlockSpec((_D, 256), lambda i: (0, 0)),
            pl.BlockSpec((_D, 256), lambda i: (0, 0)),
            pl.BlockSpec((1, 256), lambda i: (0, 0)),
            pl.BlockSpec((1, 256), lambda i: (0, 0)),
        ],
        out_specs=[
            pl.BlockSpec((_RB, 256), lambda i: (i, 0)),
            pl.BlockSpec((_RB, 256), lambda i: (i, 0)),
        ],
        out_shape=[
            jax.ShapeDtypeStruct((_NP, 256), jnp.float32),
            jax.ShapeDtypeStruct((_NP, 256), jnp.float32),
        ],
    )(dis, s_parts, hs, wmu_t, wls_t, bmu, bls)


def kernel(x, edge_index, fc_W, fc_b, conv_mu_W, conv_mu_b, conv_logstd_W,
           conv_logstd_b, addon_mu_W, addon_mu_b, addon_logstd_W,
           addon_logstd_b):
    edge_pad = jnp.pad(edge_index.astype(jnp.int32), ((0, 0), (0, _EP - _E)),
                       constant_values=_NP - 1)
    x_pad = jnp.pad(x, ((0, _NP - _N), (0, 0)))

    deg_parts = _degrees(edge_pad)
    hs, dis = _hidden_scaled(deg_parts, x_pad, fc_W.T, fc_b[None, :])
    s_parts = _aggregate(hs, edge_pad)

    wmu_t = jnp.pad(jnp.concatenate([conv_mu_W, addon_mu_W], 0),
                    ((0, 28), (0, 0))).T
    wls_t = jnp.pad(jnp.concatenate([conv_logstd_W, addon_logstd_W], 0),
                    ((0, 28), (0, 0))).T
    bmu = jnp.pad(jnp.concatenate([conv_mu_b, addon_mu_b]), (0, 28))[None, :]
    bls = jnp.pad(jnp.concatenate([conv_logstd_b, addon_logstd_b]),
                  (0, 28))[None, :]

    mu, ls = _project(dis, s_parts, hs, wmu_t, wls_t, bmu, bls)
    return mu[:_N, :228], ls[:_N, :228]


# speedup vs baseline: 30.4404x; 1.2744x over previous
import functools

import jax
import jax.numpy as jnp
from jax import lax
from jax.experimental import pallas as pl
from jax.experimental.pallas import tpu as pltpu
from jax.experimental.pallas import tpu_sc as plsc

_N = 10000
_NP = 10240
_D = 128
_E = 320000
_NW = 32
_CH = 128
_NCH = 79
_EPT = _CH * _NCH
_EP = _EPT * _NW
_RPT = _NP // 16
_RB = 1024


def _sc_mesh():
    return plsc.VectorSubcoreMesh(core_axis_name="c", subcore_axis_name="s")


def _deg_body(pk_hbm, out_hbm, idx_v, hist_v):
    c = lax.axis_index("c")
    s = lax.axis_index("s")
    w = c * 16 + s
    zero16 = jnp.zeros((16,), jnp.float32)
    ones16 = jnp.ones((16,), jnp.float32)

    def zbody(i, carry):
        hist_v[pl.ds(pl.multiple_of(i * 16, 16), 16)] = zero16
        return carry

    lax.fori_loop(0, _NP // 16, zbody, 0)

    pltpu.sync_copy(pk_hbm.at[w], idx_v)

    def body(j, carry):
        for k in range(8):
            idx = lax.shift_right_logical(idx_v[j, pl.ds(k * 16, 16)], 14)
            plsc.addupdate_scatter(hist_v, [idx], ones16)
        return carry

    lax.fori_loop(0, _NCH, body, 0)
    pltpu.sync_copy(hist_v, out_hbm.at[w])


@jax.jit
def _degrees(edge_pk):
    return pl.kernel(
        _deg_body,
        out_type=jax.ShapeDtypeStruct((_NW, _NP), jnp.float32),
        mesh=_sc_mesh(),
        scratch_types=[
            pltpu.VMEM((_NCH, _CH), jnp.int32),
            pltpu.VMEM((_NP,), jnp.float32),
        ],
        compiler_params=pltpu.CompilerParams(needs_layout_passes=False),
    )(edge_pk)


def _agg_body(hs_hbm, pk_hbm, out_hbm, acc, zbuf, rows0, rows1, pkv,
              srcb0, srcb1, dstb0, dstb1, gsem0, gsem1):
    c = lax.axis_index("c")
    s = lax.axis_index("s")
    w = c * 16 + s

    zero16 = jnp.zeros((16,), jnp.float32)
    for r in range(16):
        for k in range(8):
            zbuf[r, pl.ds(k * 16, 16)] = zero16

    rbase = pl.multiple_of(s * _RPT, 8)

    def zbody(i, carry):
        pltpu.sync_copy(
            zbuf, acc.at[pl.ds(pl.multiple_of(s * _RPT + i * 16, 16), 16)])
        return carry

    lax.fori_loop(0, _RPT // 16, zbody, 0)

    pltpu.sync_copy(pk_hbm.at[w], pkv)

    def unpack(j, srcb, dstb):
        for k in range(8):
            v = pkv[j, pl.ds(k * 16, 16)]
            srcb[pl.ds(k * 16, 16)] = lax.bitwise_and(v, 0x3FFF)
            dstb[pl.ds(k * 16, 16)] = lax.shift_right_logical(v, 14)

    plsc.subcore_barrier()

    unpack(0, srcb0, dstb0)
    pltpu.async_copy(hs_hbm.at[srcb0], rows0, gsem0)

    def body(j2, carry):
        j = j2 * 2
        unpack(j + 1, srcb1, dstb1)
        pltpu.async_copy(hs_hbm.at[srcb1], rows1, gsem1)
        pltpu.make_async_copy(hs_hbm.at[srcb0], rows0, gsem0).wait()
        pltpu.sync_copy(rows0, acc.at[dstb0], add=True)
        unpack(j + 2, srcb0, dstb0)
        pltpu.async_copy(hs_hbm.at[srcb0], rows0, gsem0)
        pltpu.make_async_copy(hs_hbm.at[srcb1], rows1, gsem1).wait()
        pltpu.sync_copy(rows1, acc.at[dstb1], add=True)
        return carry

    lax.fori_loop(0, (_NCH - 1) // 2, body, 0)
    pltpu.make_async_copy(hs_hbm.at[srcb0], rows0, gsem0).wait()
    pltpu.sync_copy(rows0, acc.at[dstb0], add=True)

    plsc.subcore_barrier()
    pltpu.sync_copy(acc.at[pl.ds(rbase, _RPT)],
                    out_hbm.at[c, pl.ds(rbase, _RPT)])


@jax.jit
def _aggregate(hs_pad, edge_pk):
    return pl.kernel(
        _agg_body,
        out_type=jax.ShapeDtypeStruct((2, _NP, _D), jnp.float32),
        mesh=_sc_mesh(),
        scratch_types=[
            pltpu.VMEM_SHARED((_NP, _D), jnp.float32),
            pltpu.VMEM((16, _D), jnp.float32),
            pltpu.VMEM((_CH, _D), jnp.float32),
            pltpu.VMEM((_CH, _D), jnp.float32),
            pltpu.VMEM((_NCH, _CH), jnp.int32),
            pltpu.VMEM((_CH,), jnp.int32),
            pltpu.VMEM((_CH,), jnp.int32),
            pltpu.VMEM((_CH,), jnp.int32),
            pltpu.VMEM((_CH,), jnp.int32),
            pltpu.SemaphoreType.DMA,
            pltpu.SemaphoreType.DMA,
        ],
        compiler_params=pltpu.CompilerParams(needs_layout_passes=False),
    )(hs_pad, edge_pk)


def _hs_body(degp_ref, x_ref, w_ref, b_ref, hs_ref, dis_ref):
    deg = jnp.sum(degp_ref[...], axis=0, keepdims=True) + 1.0
    dis = lax.rsqrt(deg).T
    h = jnp.maximum(
        jnp.dot(x_ref[...], w_ref[...], preferred_element_type=jnp.float32)
        + b_ref[...], 0.0)
    hs_ref[...] = h * dis
    dis_ref[...] = dis


@jax.jit
def _hidden_scaled(deg_parts, x_pad, fc_Wt, fc_b2):
    return pl.pallas_call(
        _hs_body,
        grid=(_NP // _RB,),
        in_specs=[
            pl.BlockSpec((_NW, _RB), lambda i: (0, i)),
            pl.BlockSpec((_RB, _D), lambda i: (i, 0)),
            pl.BlockSpec((_D, _D), lambda i: (0, 0)),
            pl.BlockSpec((1, _D), lambda i: (0, 0)),
        ],
        out_specs=[
            pl.BlockSpec((_RB, _D), lambda i: (i, 0)),
            pl.BlockSpec((_RB, 1), lambda i: (i, 0)),
        ],
        out_shape=[
            jax.ShapeDtypeStruct((_NP, _D), jnp.float32),
            jax.ShapeDtypeStruct((_NP, 1), jnp.float32),
        ],
    )(deg_parts, x_pad, fc_Wt, fc_b2)


def _proj_body(dis_ref, sp_ref, hs_ref, wmu_ref, wls_ref, bmu_ref, bls_ref,
               mu_ref, ls_ref):
    ssum = sp_ref[0] + sp_ref[1]
    agg = dis_ref[...] * (ssum + hs_ref[...])
    mu_ref[...] = jnp.dot(agg, wmu_ref[...],
                          preferred_element_type=jnp.float32) + bmu_ref[...]
    ls_ref[...] = jnp.dot(agg, wls_ref[...],
                          preferred_element_type=jnp.float32) + bls_ref[...]


@jax.jit
def _project(dis, s_parts, hs, wmu_t, wls_t, bmu, bls):
    return pl.pallas_call(
        _proj_body,
        grid=(_NP // _RB,),
        in_specs=[
            pl.BlockSpec((_RB, 1), lambda i: (i, 0)),
            pl.BlockSpec((2, _RB, _D), lambda i: (0, i, 0)),
            pl.BlockSpec((_RB, _D), lambda i: (i, 0)),
            pl.BlockSpec((_D, 256), lambda i: (0, 0)),
            pl.BlockSpec((_D, 256), lambda i: (0, 0)),
            pl.BlockSpec((1, 256), lambda i: (0, 0)),
            pl.BlockSpec((1, 256), lambda i: (0, 0)),
        ],
        out_specs=[
            pl.BlockSpec((_RB, 256), lambda i: (i, 0)),
            pl.BlockSpec((_RB, 256), lambda i: (i, 0)),
        ],
        out_shape=[
            jax.ShapeDtypeStruct((_NP, 256), jnp.float32),
            jax.ShapeDtypeStruct((_NP, 256), jnp.float32),
        ],
    )(dis, s_parts, hs, wmu_t, wls_t, bmu, bls)


def kernel(x, edge_index, fc_W, fc_b, conv_mu_W, conv_mu_b, conv_logstd_W,
           conv_logstd_b, addon_mu_W, addon_mu_b, addon_logstd_W,
           addon_logstd_b):
    edge_pad = jnp.pad(edge_index.astype(jnp.int32), ((0, 0), (0, _EP - _E)),
                       constant_values=_NP - 1)
    edge_pk = jnp.bitwise_or(
        edge_pad[0], jnp.left_shift(edge_pad[1], 14)).reshape(_NW, _NCH, _CH)
    x_pad = jnp.pad(x, ((0, _NP - _N), (0, 0)))

    deg_parts = _degrees(edge_pk)
    hs, dis = _hidden_scaled(deg_parts, x_pad, fc_W.T, fc_b[None, :])
    s_parts = _aggregate(hs, edge_pk)

    wmu_t = jnp.pad(jnp.concatenate([conv_mu_W, addon_mu_W], 0),
                    ((0, 28), (0, 0))).T
    wls_t = jnp.pad(jnp.concatenate([conv_logstd_W, addon_logstd_W], 0),
                    ((0, 28), (0, 0))).T
    bmu = jnp.pad(jnp.concatenate([conv_mu_b, addon_mu_b]), (0, 28))[None, :]
    bls = jnp.pad(jnp.concatenate([conv_logstd_b, addon_logstd_b]),
                  (0, 28))[None, :]

    mu, ls = _project(dis, s_parts, hs, wmu_t, wls_t, bmu, bls)
    return mu[:_N, :228], ls[:_N, :228]


# speedup vs baseline: 37.0795x; 1.2181x over previous
import functools

import jax
import jax.numpy as jnp
from jax import lax
from jax.experimental import pallas as pl
from jax.experimental.pallas import tpu as pltpu
from jax.experimental.pallas import tpu_sc as plsc

_N = 10000
_NP = 10240
_D = 128
_E = 320000
_NW = 32
_CH = 128
_NCH = 79
_EPT = _CH * _NCH
_EP = _EPT * _NW
_RPT = _NP // 16
_RB = 1024


def _sc_mesh():
    return plsc.VectorSubcoreMesh(core_axis_name="c", subcore_axis_name="s")


def _deg_body(pk_hbm, out_hbm, idx_v, hist_v):
    c = lax.axis_index("c")
    s = lax.axis_index("s")
    w = c * 16 + s
    zero16 = jnp.zeros((16,), jnp.float32)
    ones16 = jnp.ones((16,), jnp.float32)

    def zbody(i, carry):
        hist_v[pl.ds(pl.multiple_of(i * 16, 16), 16)] = zero16
        return carry

    lax.fori_loop(0, _NP // 16, zbody, 0)

    pltpu.sync_copy(pk_hbm.at[w], idx_v)

    def body(j, carry):
        for k in range(8):
            idx = lax.shift_right_logical(idx_v[j, pl.ds(k * 16, 16)], 14)
            plsc.addupdate_scatter(hist_v, [idx], ones16)
        return carry

    lax.fori_loop(0, _NCH, body, 0)
    pltpu.sync_copy(hist_v, out_hbm.at[w])


@jax.jit
def _degrees(edge_pk):
    return pl.kernel(
        _deg_body,
        out_type=jax.ShapeDtypeStruct((_NW, _NP), jnp.float32),
        mesh=_sc_mesh(),
        scratch_types=[
            pltpu.VMEM((_NCH, _CH), jnp.int32),
            pltpu.VMEM((_NP,), jnp.float32),
        ],
        compiler_params=pltpu.CompilerParams(needs_layout_passes=False),
    )(edge_pk)


def _agg_body(hs_hbm, pk_hbm, out_hbm, acc, zbuf, rows0, rows1, pkv,
              srcb0, srcb1, dstb0, dstb1, gsem0, gsem1):
    c = lax.axis_index("c")
    s = lax.axis_index("s")
    w = c * 16 + s

    zero16 = jnp.zeros((16,), jnp.float32)
    for r in range(16):
        for k in range(8):
            zbuf[r, pl.ds(k * 16, 16)] = zero16

    rbase = pl.multiple_of(s * _RPT, 8)

    def zbody(i, carry):
        pltpu.sync_copy(
            zbuf, acc.at[pl.ds(pl.multiple_of(s * _RPT + i * 16, 16), 16)])
        return carry

    lax.fori_loop(0, _RPT // 16, zbody, 0)

    pltpu.sync_copy(pk_hbm.at[w], pkv)

    def unpack(j, srcb, dstb):
        for k in range(8):
            v = pkv[j, pl.ds(k * 16, 16)]
            srcb[pl.ds(k * 16, 16)] = lax.bitwise_and(v, 0x3FFF)
            dstb[pl.ds(k * 16, 16)] = lax.shift_right_logical(v, 14)

    plsc.subcore_barrier()

    unpack(0, srcb0, dstb0)
    pltpu.async_copy(hs_hbm.at[srcb0], rows0, gsem0)

    def body(j2, carry):
        j = j2 * 2
        unpack(j + 1, srcb1, dstb1)
        pltpu.async_copy(hs_hbm.at[srcb1], rows1, gsem1)
        pltpu.make_async_copy(hs_hbm.at[srcb0], rows0, gsem0).wait()
        pltpu.sync_copy(rows0, acc.at[dstb0], add=True)
        unpack(j + 2, srcb0, dstb0)
        pltpu.async_copy(hs_hbm.at[srcb0], rows0, gsem0)
        pltpu.make_async_copy(hs_hbm.at[srcb1], rows1, gsem1).wait()
        pltpu.sync_copy(rows1, acc.at[dstb1], add=True)
        return carry

    lax.fori_loop(0, (_NCH - 1) // 2, body, 0)
    pltpu.make_async_copy(hs_hbm.at[srcb0], rows0, gsem0).wait()
    pltpu.sync_copy(rows0, acc.at[dstb0], add=True)

    plsc.subcore_barrier()
    pltpu.sync_copy(acc.at[pl.ds(rbase, _RPT)],
                    out_hbm.at[c, pl.ds(rbase, _RPT)])


@jax.jit
def _aggregate(hs_pad, edge_pk):
    return pl.kernel(
        _agg_body,
        out_type=jax.ShapeDtypeStruct((2, _NP, _D), jnp.float32),
        mesh=_sc_mesh(),
        scratch_types=[
            pltpu.VMEM_SHARED((_NP, _D), jnp.float32),
            pltpu.VMEM((16, _D), jnp.float32),
            pltpu.VMEM((_CH, _D), jnp.float32),
            pltpu.VMEM((_CH, _D), jnp.float32),
            pltpu.VMEM((_NCH, _CH), jnp.int32),
            pltpu.VMEM((_CH,), jnp.int32),
            pltpu.VMEM((_CH,), jnp.int32),
            pltpu.VMEM((_CH,), jnp.int32),
            pltpu.VMEM((_CH,), jnp.int32),
            pltpu.SemaphoreType.DMA,
            pltpu.SemaphoreType.DMA,
        ],
        compiler_params=pltpu.CompilerParams(needs_layout_passes=False),
    )(hs_pad, edge_pk)


def _hs_body(degp_ref, x_ref, w_ref, b_ref, hs_ref, dis_ref):
    deg = jnp.sum(degp_ref[...], axis=0, keepdims=True) + 1.0
    dis = lax.rsqrt(deg).T
    h = jnp.maximum(
        jnp.dot(x_ref[...], w_ref[...], preferred_element_type=jnp.float32)
        + b_ref[...], 0.0)
    hs_ref[...] = h * dis
    dis_ref[...] = dis


@jax.jit
def _hidden_scaled(deg_parts, x_pad, fc_Wt, fc_b2):
    return pl.pallas_call(
        _hs_body,
        grid=(_NP // _RB,),
        in_specs=[
            pl.BlockSpec((_NW, _RB), lambda i: (0, i)),
            pl.BlockSpec((_RB, _D), lambda i: (i, 0)),
            pl.BlockSpec((_D, _D), lambda i: (0, 0)),
            pl.BlockSpec((1, _D), lambda i: (0, 0)),
        ],
        out_specs=[
            pl.BlockSpec((_RB, _D), lambda i: (i, 0)),
            pl.BlockSpec((_RB, 1), lambda i: (i, 0)),
        ],
        out_shape=[
            jax.ShapeDtypeStruct((_NP, _D), jnp.float32),
            jax.ShapeDtypeStruct((_NP, 1), jnp.float32),
        ],
    )(deg_parts, x_pad, fc_Wt, fc_b2)


def _proj_body(dis_ref, sp_ref, hs_ref, wmu_ref, wls_ref, bmu_ref, bls_ref,
               mu_ref, ls_ref):
    ssum = sp_ref[0] + sp_ref[1]
    agg = dis_ref[...] * (ssum + hs_ref[...])
    mu_ref[...] = jnp.dot(agg, wmu_ref[...],
                          preferred_element_type=jnp.float32) + bmu_ref[...]
    ls_ref[...] = jnp.dot(agg, wls_ref[...],
                          preferred_element_type=jnp.float32) + bls_ref[...]


@jax.jit
def _project(dis, s_parts, hs, wmu_t, wls_t, bmu, bls):
    rb = 1000
    return pl.pallas_call(
        _proj_body,
        grid=(_N // rb,),
        in_specs=[
            pl.BlockSpec((rb, 1), lambda i: (i, 0)),
            pl.BlockSpec((2, rb, _D), lambda i: (0, i, 0)),
            pl.BlockSpec((rb, _D), lambda i: (i, 0)),
            pl.BlockSpec((_D, 228), lambda i: (0, 0)),
            pl.BlockSpec((_D, 228), lambda i: (0, 0)),
            pl.BlockSpec((1, 228), lambda i: (0, 0)),
            pl.BlockSpec((1, 228), lambda i: (0, 0)),
        ],
        out_specs=[
            pl.BlockSpec((rb, 228), lambda i: (i, 0)),
            pl.BlockSpec((rb, 228), lambda i: (i, 0)),
        ],
        out_shape=[
            jax.ShapeDtypeStruct((_N, 228), jnp.float32),
            jax.ShapeDtypeStruct((_N, 228), jnp.float32),
        ],
    )(dis, s_parts, hs, wmu_t, wls_t, bmu, bls)


def kernel(x, edge_index, fc_W, fc_b, conv_mu_W, conv_mu_b, conv_logstd_W,
           conv_logstd_b, addon_mu_W, addon_mu_b, addon_logstd_W,
           addon_logstd_b):
    edge_pad = jnp.pad(edge_index.astype(jnp.int32), ((0, 0), (0, _EP - _E)),
                       constant_values=_NP - 1)
    edge_pk = jnp.bitwise_or(
        edge_pad[0], jnp.left_shift(edge_pad[1], 14)).reshape(_NW, _NCH, _CH)
    x_pad = jnp.pad(x, ((0, _NP - _N), (0, 0)))

    deg_parts = _degrees(edge_pk)
    hs, dis = _hidden_scaled(deg_parts, x_pad, fc_W.T, fc_b[None, :])
    s_parts = _aggregate(hs, edge_pk)

    wmu_t = jnp.concatenate([conv_mu_W, addon_mu_W], 0).T
    wls_t = jnp.concatenate([conv_logstd_W, addon_logstd_W], 0).T
    bmu = jnp.concatenate([conv_mu_b, addon_mu_b])[None, :]
    bls = jnp.concatenate([conv_logstd_b, addon_logstd_b])[None, :]

    return _project(dis, s_parts, hs, wmu_t, wls_t, bmu, bls)
